# baseline (speedup 1.0000x reference)
import functools
import math

import jax
import jax.numpy as jnp
from jax import lax
from jax.experimental import pallas as pl
from jax.experimental.pallas import tpu as pltpu
from jax.experimental.pallas import tpu_sc as plsc

_LANES = 16
_NUM_WORKERS = 32
_GROUP = 16


def kernel(input_ids, token_table, pos_table):
    B, S = input_ids.shape
    V, D = token_table.shape
    N = B * S
    scale = math.sqrt(float(D))
    s_per_w = S // _NUM_WORKERS
    nvec = D // _LANES
    ngrp = s_per_w // _GROUP
    grows = B * _GROUP

    mesh = plsc.VectorSubcoreMesh(core_axis_name="c", subcore_axis_name="s")

    @functools.partial(
        pl.kernel,
        mesh=mesh,
        out_type=jax.ShapeDtypeStruct((N, D), jnp.float32),
        scratch_types=[
            pltpu.VMEM((B, s_per_w), jnp.int32),
            pltpu.VMEM((ngrp, grows), jnp.int32),
            pltpu.VMEM((s_per_w, D), jnp.float32),
            pltpu.VMEM((grows, D), jnp.float32),
            pltpu.VMEM((grows, D), jnp.float32),
            pltpu.SemaphoreType.DMA,
            pltpu.SemaphoreType.DMA,
            pltpu.SemaphoreType.DMA,
            pltpu.SemaphoreType.DMA,
            pltpu.SemaphoreType.DMA,
            pltpu.SemaphoreType.DMA,
        ],
    )
    def body(ids_hbm, tok_hbm, pos_hbm, out_hbm, idx_v, gidx_v, pos_v, t0, t1,
             g0, g1, o0, o1, isem, psem):
        wid = lax.axis_index("s") * 2 + lax.axis_index("c")
        s0 = wid * s_per_w
        idx_cps = [
            pltpu.async_copy(ids_hbm.at[pl.ds(b * S + s0, s_per_w)],
                             idx_v.at[b], isem)
            for b in range(B)
        ]
        pos_cp = pltpu.async_copy(pos_hbm.at[pl.ds(s0, s_per_w)], pos_v, psem)
        for cp in idx_cps:
            cp.wait()
        for g in range(ngrp):
            for b in range(B):
                gidx_v[g, pl.ds(b * _GROUP, _GROUP)] = (
                    idx_v[b, pl.ds(g * _GROUP, _GROUP)])

        tbufs = [t0, t1]
        gsems = [g0, g1]
        osems = [o0, o1]
        gathers = [None, None]
        stores = [[], []]

        def start_gather(g):
            slot = g % 2
            gathers[slot] = pltpu.async_copy(
                tok_hbm.at[gidx_v.at[g]], tbufs[slot], gsems[slot])

        start_gather(0)
        pos_cp.wait()
        for g in range(ngrp):
            slot = g % 2
            nslot = (g + 1) % 2
            if g + 1 < ngrp:
                for st in stores[nslot]:
                    st.wait()
                stores[nslot] = []
                start_gather(g + 1)
            gathers[slot].wait()
            buf = tbufs[slot]
            poff = g * _GROUP

            def vec(k, _, buf=buf, poff=poff):
                sl = pl.ds(k * _LANES, _LANES)
                for i in range(_GROUP):
                    p = pos_v[poff + i, sl]
                    for b in range(B):
                        r = b * _GROUP + i
                        buf[r, sl] = buf[r, sl] * scale + p
                return 0

            lax.fori_loop(0, nvec, vec, 0)
            for b in range(B):
                stores[slot].append(pltpu.async_copy(
                    tbufs[slot].at[pl.ds(b * _GROUP, _GROUP)],
                    out_hbm.at[pl.ds(b * S + s0 + poff, _GROUP)],
                    osems[slot]))
        for sl in stores:
            for st in sl:
                st.wait()

    out = body(input_ids.reshape(N), token_table, pos_table)
    return out.reshape(B, S, D)


# speedup vs baseline: 1.0911x; 1.0911x over previous
import functools
import math

import jax
import jax.numpy as jnp
from jax import lax
from jax.experimental import pallas as pl
from jax.experimental.pallas import tpu as pltpu
from jax.experimental.pallas import tpu_sc as plsc

_LANES = 16
_NUM_WORKERS = 32
_HALF = 32


def kernel(input_ids, token_table, pos_table):
    B, S = input_ids.shape
    V, D = token_table.shape
    N = B * S
    scale = math.sqrt(float(D))
    s_per_w = S // _NUM_WORKERS
    nvec = D // _LANES

    mesh = plsc.VectorSubcoreMesh(core_axis_name="c", subcore_axis_name="s")

    @functools.partial(
        pl.kernel,
        mesh=mesh,
        out_type=jax.ShapeDtypeStruct((N, D), jnp.float32),
        scratch_types=[
            pltpu.VMEM((B, s_per_w), jnp.int32),
            pltpu.VMEM((s_per_w, D), jnp.float32),
            pltpu.VMEM((s_per_w, D), jnp.float32),
            pltpu.VMEM((s_per_w, D), jnp.float32),
            pltpu.SemaphoreType.DMA,
            pltpu.SemaphoreType.DMA,
            pltpu.SemaphoreType.DMA,
            pltpu.SemaphoreType.DMA,
            pltpu.SemaphoreType.DMA,
            pltpu.SemaphoreType.DMA,
        ],
    )
    def body(ids_hbm, tok_hbm, pos_hbm, out_hbm, idx_v, pos_v, t0, t1,
             g0, g1, o0, o1, isem, psem):
        wid = lax.axis_index("s") * 2 + lax.axis_index("c")
        s0 = wid * s_per_w
        idx_cps = [
            pltpu.async_copy(ids_hbm.at[pl.ds(b * S + s0, s_per_w)],
                             idx_v.at[b], isem)
            for b in range(B)
        ]
        pos_cp = pltpu.async_copy(pos_hbm.at[pl.ds(s0, s_per_w)], pos_v, psem)
        for cp in idx_cps:
            cp.wait()

        tbufs = [t0, t1]
        gsems = [g0, g1]
        osems = [o0, o1]
        gathers = [None, None]
        stores = [[], []]

        def sweep_half(buf, h):
            def row(i, _):
                for k in range(nvec):
                    sl = pl.ds(k * _LANES, _LANES)
                    r = h * _HALF + i
                    buf[r, sl] = buf[r, sl] * scale + pos_v[r, sl]
                return 0
            lax.fori_loop(0, _HALF, row, 0)

        gathers[0] = pltpu.async_copy(tok_hbm.at[idx_v.at[0]], t0, g0)
        pos_cp.wait()
        for b in range(B):
            cur = b % 2
            nxt = (b + 1) % 2
            gathers[cur].wait()
            buf = tbufs[cur]
            sweep_half(buf, 0)
            stores[cur].append(pltpu.async_copy(
                buf.at[pl.ds(0, _HALF)],
                out_hbm.at[pl.ds(b * S + s0, _HALF)], osems[cur]))
            if b + 1 < B:
                for st in stores[nxt]:
                    st.wait()
                stores[nxt] = []
                gathers[nxt] = pltpu.async_copy(
                    tok_hbm.at[idx_v.at[b + 1]], tbufs[nxt], gsems[nxt])
            sweep_half(buf, 1)
            stores[cur].append(pltpu.async_copy(
                buf.at[pl.ds(_HALF, _HALF)],
                out_hbm.at[pl.ds(b * S + s0 + _HALF, _HALF)], osems[cur]))
        for sl in stores:
            for st in sl:
                st.wait()

    out = body(input_ids.reshape(N), token_table, pos_table)
    return out.reshape(B, S, D)
